# baseline (speedup 1.0000x reference)
import functools

import jax
import jax.numpy as jnp
from jax import lax
from jax.experimental import pallas as pl
from jax.experimental.pallas import tpu as pltpu
from jax.experimental.pallas import tpu_sc as plsc

N0 = 10000
NP = 10240
E0 = 320000
B = 128
NW = 32
NS = 16
NC = 2
G = 80
NSL = 4
EP = NW * G * B
RPT = NP // NS

_mesh = functools.partial(
    plsc.VectorSubcoreMesh, core_axis_name="c", subcore_axis_name="s"
)


def _make_deg_kernel():
    @functools.partial(
        pl.kernel,
        mesh=_mesh(),
        out_type=jax.ShapeDtypeStruct((NC, NP), jnp.float32),
        scratch_types=[
            pltpu.VMEM((G + NSL, B), jnp.int32),
            pltpu.VMEM((128,), jnp.float32),
            pltpu.VMEM((B,), jnp.float32),
            pltpu.VMEM_SHARED((NP,), jnp.float32),
        ],
    )
    def deg_kernel(dst_hbm, out_hbm, dst_v, zbuf, ones_v, acc):
        c = lax.axis_index("c")
        s = lax.axis_index("s")
        wid = s * NC + c
        pltpu.sync_copy(dst_hbm.at[wid], dst_v)
        for i in range(8):
            zbuf[pl.ds(i * 16, 16)] = jnp.zeros((16,), jnp.float32)
        for i in range(B // 16):
            ones_v[pl.ds(i * 16, 16)] = jnp.ones((16,), jnp.float32)
        base = s * RPT
        for off in range(0, RPT, 128):
            pltpu.sync_copy(zbuf, acc.at[pl.ds(base + off, 128)])
        plsc.subcore_barrier()

        def body(g, carry):
            pltpu.sync_copy(ones_v, acc.at[dst_v.at[g]], add=True)
            return carry

        lax.fori_loop(0, G, body, None)
        plsc.subcore_barrier()
        for off in range(0, RPT, 128):
            pltpu.sync_copy(
                acc.at[pl.ds(base + off, 128)], out_hbm.at[c, pl.ds(base + off, 128)]
            )

    return deg_kernel


def _make_scatter_kernel(D):
    @functools.partial(
        pl.kernel,
        mesh=_mesh(),
        out_type=jax.ShapeDtypeStruct((NC, NP, D), jnp.float32),
        scratch_types=[
            pltpu.VMEM((NSL, B), jnp.int32),
            pltpu.VMEM((NSL, B), jnp.int32),
            pltpu.VMEM((B, D), jnp.float32),
            pltpu.VMEM((B, D), jnp.float32),
            pltpu.VMEM_SHARED((NP, D), jnp.float32),
            pltpu.SemaphoreType.DMA,
            pltpu.SemaphoreType.DMA,
            pltpu.SemaphoreType.DMA,
            pltpu.SemaphoreType.DMA,
            pltpu.SemaphoreType.DMA,
            pltpu.SemaphoreType.DMA,
        ],
    )
    def scatter_kernel(
        table_hbm, src_hbm, dst_hbm, out_hbm,
        sidx, didx, rbuf0, rbuf1, acc,
        gsem0, gsem1, isem0, isem1, isem2, isem3,
    ):
        c = lax.axis_index("c")
        s = lax.axis_index("s")
        wid = s * NC + c
        rbufs = (rbuf0, rbuf1)
        gsems = (gsem0, gsem1)
        isems = (isem0, isem1, isem2, isem3)

        def zrow(r, carry):
            for i in range(D // 16):
                rbuf0[r, pl.ds(i * 16, 16)] = jnp.zeros((16,), jnp.float32)
            return carry

        lax.fori_loop(0, B, zrow, None)
        base = s * RPT
        for off in range(0, RPT, B):
            pltpu.sync_copy(rbuf0, acc.at[pl.ds(base + off, B)])
        plsc.subcore_barrier()

        def prefetch_idx(b2, g):
            pltpu.async_copy(src_hbm.at[wid, g], sidx.at[b2], isems[b2])
            pltpu.async_copy(dst_hbm.at[wid, g], didx.at[b2], isems[b2])

        def wait_idx(b2, g):
            pltpu.make_async_copy(src_hbm.at[wid, g], sidx.at[b2], isems[b2]).wait()
            pltpu.make_async_copy(dst_hbm.at[wid, g], didx.at[b2], isems[b2]).wait()

        for b2 in range(NSL):
            prefetch_idx(b2, b2)
        wait_idx(0, 0)
        pltpu.async_copy(table_hbm.at[sidx.at[0]], rbuf0, gsem0)

        def body(i, carry):
            for b in range(NSL):
                g = i * NSL + b
                nb = (b + 1) % NSL
                wait_idx(nb, g + 1)
                pltpu.async_copy(
                    table_hbm.at[sidx.at[nb]], rbufs[(b + 1) % 2], gsems[(b + 1) % 2]
                )
                pltpu.make_async_copy(
                    table_hbm.at[sidx.at[b]], rbufs[b % 2], gsems[b % 2]
                ).wait()
                pltpu.sync_copy(rbufs[b % 2], acc.at[didx.at[b]], add=True)
                prefetch_idx(b, g + NSL)
            return carry

        lax.fori_loop(0, G // NSL, body, None)
        pltpu.make_async_copy(table_hbm.at[sidx.at[0]], rbuf0, gsem0).wait()
        for b2 in range(1, NSL):
            wait_idx(b2, G + b2)
        plsc.subcore_barrier()
        for off in range(0, RPT, B):
            pltpu.sync_copy(
                acc.at[pl.ds(base + off, B)],
                out_hbm.at[c, pl.ds(base + off, B)],
            )

    return scatter_kernel


_deg_kernel = _make_deg_kernel()
_scatter128 = _make_scatter_kernel(128)


def _tc1_body(degp_ref, x_ref, dinv_ref, xp_ref):
    deg = degp_ref[0] + degp_ref[1] + 1.0
    dinv = lax.rsqrt(deg)
    dinv_ref[...] = dinv
    xp_ref[...] = x_ref[...] * dinv


def _tc2_body(parts_ref, xp_ref, dinv_ref, w1_ref, b1_ref, w2_ref, zp_ref):
    dinv = dinv_ref[...]
    p = (parts_ref[0] + parts_ref[1] + xp_ref[...]) * dinv
    h = jnp.dot(p, w1_ref[...], preferred_element_type=jnp.float32) + b1_ref[...]
    h = jnp.maximum(h, 0.0)
    z = jnp.dot(h, w2_ref[...], preferred_element_type=jnp.float32)
    zp_ref[...] = z * dinv


def _tc3_body(parts_ref, zp_ref, dinv_ref, b2_ref, out_ref):
    d_out = out_ref.shape[1]
    out_ref[...] = (
        parts_ref[0][:, :d_out] + parts_ref[1][:, :d_out] + zp_ref[:, :d_out]
    ) * dinv_ref[...] + b2_ref[...]


def kernel(x, edge_index, W1, b1, W2, b2):
    d_in = x.shape[1]
    d_hid = W1.shape[1]
    d_out = W2.shape[1]

    pad = EP - E0
    extra = jnp.full((NW, NSL, B), N0, jnp.int32)
    src_p = jnp.concatenate(
        [
            jnp.concatenate([edge_index[0], jnp.full((pad,), N0, jnp.int32)]).reshape(
                NW, G, B
            ),
            extra,
        ],
        axis=1,
    )
    dst_p = jnp.concatenate(
        [
            jnp.concatenate([edge_index[1], jnp.full((pad,), N0, jnp.int32)]).reshape(
                NW, G, B
            ),
            extra,
        ],
        axis=1,
    )
    x_pad = jnp.concatenate([x, jnp.zeros((NP - N0, d_in), x.dtype)])

    deg_parts = _deg_kernel(dst_p)

    dinv, xp = pl.pallas_call(
        _tc1_body,
        out_shape=(
            jax.ShapeDtypeStruct((NP, 1), jnp.float32),
            jax.ShapeDtypeStruct((NP, d_in), jnp.float32),
        ),
    )(deg_parts.reshape(NC, NP, 1), x_pad)

    parts1 = _scatter128(xp, src_p, dst_p)

    DP = 128
    W2p = jnp.concatenate([W2, jnp.zeros((d_hid, DP - d_out), W2.dtype)], axis=1)
    R = 1024
    zp = pl.pallas_call(
        _tc2_body,
        grid=(NP // R,),
        in_specs=[
            pl.BlockSpec((NC, R, d_in), lambda i: (0, i, 0)),
            pl.BlockSpec((R, d_in), lambda i: (i, 0)),
            pl.BlockSpec((R, 1), lambda i: (i, 0)),
            pl.BlockSpec((d_in, d_hid), lambda i: (0, 0)),
            pl.BlockSpec((1, d_hid), lambda i: (0, 0)),
            pl.BlockSpec((d_hid, DP), lambda i: (0, 0)),
        ],
        out_specs=pl.BlockSpec((R, DP), lambda i: (i, 0)),
        out_shape=jax.ShapeDtypeStruct((NP, DP), jnp.float32),
    )(parts1, xp, dinv, W1, b1.reshape(1, d_hid), W2p)

    parts2 = _scatter128(zp, src_p, dst_p)

    out_pad = pl.pallas_call(
        _tc3_body,
        grid=(NP // R,),
        in_specs=[
            pl.BlockSpec((NC, R, DP), lambda i: (0, i, 0)),
            pl.BlockSpec((R, DP), lambda i: (i, 0)),
            pl.BlockSpec((R, 1), lambda i: (i, 0)),
            pl.BlockSpec((1, d_out), lambda i: (0, 0)),
        ],
        out_specs=pl.BlockSpec((R, d_out), lambda i: (i, 0)),
        out_shape=jax.ShapeDtypeStruct((NP, d_out), jnp.float32),
    )(parts2, zp, dinv, b2.reshape(1, d_out))

    return out_pad[:N0]


# speedup vs baseline: 3.0311x; 3.0311x over previous
import functools

import jax
import jax.numpy as jnp
from jax import lax
from jax.experimental import pallas as pl
from jax.experimental.pallas import tpu as pltpu
from jax.experimental.pallas import tpu_sc as plsc

N0 = 10000
NP = 10240
E0 = 320000
B = 128
NW = 32
NS = 16
NC = 2
G = 80
EP = NW * G * B
RPT = NP // NS

_mesh = functools.partial(
    plsc.VectorSubcoreMesh, core_axis_name="c", subcore_axis_name="s"
)


def _make_deg_kernel():
    @functools.partial(
        pl.kernel,
        mesh=_mesh(),
        out_type=jax.ShapeDtypeStruct((NC, NP), jnp.float32),
        scratch_types=[
            pltpu.VMEM((G, B), jnp.int32),
            pltpu.VMEM((128,), jnp.float32),
            pltpu.VMEM((B,), jnp.float32),
            pltpu.VMEM_SHARED((NP,), jnp.float32),
        ],
    )
    def deg_kernel(dst_hbm, out_hbm, dst_v, zbuf, ones_v, acc):
        c = lax.axis_index("c")
        s = lax.axis_index("s")
        wid = s * NC + c
        pltpu.sync_copy(dst_hbm.at[wid], dst_v)
        for i in range(8):
            zbuf[pl.ds(i * 16, 16)] = jnp.zeros((16,), jnp.float32)
        for i in range(B // 16):
            ones_v[pl.ds(i * 16, 16)] = jnp.ones((16,), jnp.float32)
        base = s * RPT
        for off in range(0, RPT, 128):
            pltpu.sync_copy(zbuf, acc.at[pl.ds(base + off, 128)])
        plsc.subcore_barrier()

        def body(g, carry):
            pltpu.sync_copy(ones_v, acc.at[dst_v.at[g]], add=True)
            return carry

        lax.fori_loop(0, G, body, None)
        plsc.subcore_barrier()
        for off in range(0, RPT, 128):
            pltpu.sync_copy(
                acc.at[pl.ds(base + off, 128)], out_hbm.at[c, pl.ds(base + off, 128)]
            )

    return deg_kernel


def _make_scatter_kernel(D):
    @functools.partial(
        pl.kernel,
        mesh=_mesh(),
        out_type=jax.ShapeDtypeStruct((NC, NP, D), jnp.float32),
        scratch_types=[
            pltpu.VMEM((G, B), jnp.int32),
            pltpu.VMEM((G, B), jnp.int32),
            pltpu.VMEM((B, D), jnp.float32),
            pltpu.VMEM_SHARED((NP, D), jnp.float32),
            pltpu.SemaphoreType.DMA,
        ],
    )
    def scatter_kernel(table_hbm, src_hbm, dst_hbm, out_hbm, src_v, dst_v, rbuf, acc, sem):
        c = lax.axis_index("c")
        s = lax.axis_index("s")
        wid = s * NC + c
        pltpu.sync_copy(src_hbm.at[wid], src_v)
        pltpu.sync_copy(dst_hbm.at[wid], dst_v)

        def zrow(r, carry):
            for i in range(D // 16):
                rbuf[r, pl.ds(i * 16, 16)] = jnp.zeros((16,), jnp.float32)
            return carry

        lax.fori_loop(0, B, zrow, None)
        base = s * RPT
        for off in range(0, RPT, B):
            pltpu.sync_copy(rbuf, acc.at[pl.ds(base + off, B)])
        plsc.subcore_barrier()

        def body(g, carry):
            pltpu.async_copy(table_hbm.at[src_v.at[g]], rbuf, sem).wait()
            pltpu.sync_copy(rbuf, acc.at[dst_v.at[g]], add=True)
            return carry

        lax.fori_loop(0, G, body, None)
        plsc.subcore_barrier()
        for off in range(0, RPT, B):
            pltpu.sync_copy(
                acc.at[pl.ds(base + off, B)],
                out_hbm.at[c, pl.ds(base + off, B)],
            )

    return scatter_kernel


_deg_kernel = _make_deg_kernel()
_scatter128 = _make_scatter_kernel(128)


def _tc1_body(degp_ref, x_ref, dinv_ref, xp_ref):
    deg = degp_ref[0] + degp_ref[1] + 1.0
    dinv = lax.rsqrt(deg)
    dinv_ref[...] = dinv
    xp_ref[...] = x_ref[...] * dinv


def _tc2_body(parts_ref, xp_ref, dinv_ref, w1_ref, b1_ref, w2_ref, zp_ref):
    dinv = dinv_ref[...]
    p = (parts_ref[0] + parts_ref[1] + xp_ref[...]) * dinv
    h = jnp.dot(p, w1_ref[...], preferred_element_type=jnp.float32) + b1_ref[...]
    h = jnp.maximum(h, 0.0)
    z = jnp.dot(h, w2_ref[...], preferred_element_type=jnp.float32)
    zp_ref[...] = z * dinv


def _tc3_body(parts_ref, zp_ref, dinv_ref, b2_ref, out_ref):
    d_out = out_ref.shape[1]
    out_ref[...] = (
        parts_ref[0][:, :d_out] + parts_ref[1][:, :d_out] + zp_ref[:, :d_out]
    ) * dinv_ref[...] + b2_ref[...]


def kernel(x, edge_index, W1, b1, W2, b2):
    d_in = x.shape[1]
    d_hid = W1.shape[1]
    d_out = W2.shape[1]

    pad = EP - E0
    pad_idx = N0 + (jnp.arange(pad, dtype=jnp.int32) % (NP - N0))
    src_p = jnp.concatenate([edge_index[0], pad_idx]).reshape(NW, G, B)
    dst_p = jnp.concatenate([edge_index[1], pad_idx]).reshape(NW, G, B)
    x_pad = jnp.concatenate([x, jnp.zeros((NP - N0, d_in), x.dtype)])

    deg_parts = _deg_kernel(dst_p)

    dinv, xp = pl.pallas_call(
        _tc1_body,
        out_shape=(
            jax.ShapeDtypeStruct((NP, 1), jnp.float32),
            jax.ShapeDtypeStruct((NP, d_in), jnp.float32),
        ),
    )(deg_parts.reshape(NC, NP, 1), x_pad)

    parts1 = _scatter128(xp, src_p, dst_p)

    DP = 128
    W2p = jnp.concatenate([W2, jnp.zeros((d_hid, DP - d_out), W2.dtype)], axis=1)
    R = 1024
    zp = pl.pallas_call(
        _tc2_body,
        grid=(NP // R,),
        in_specs=[
            pl.BlockSpec((NC, R, d_in), lambda i: (0, i, 0)),
            pl.BlockSpec((R, d_in), lambda i: (i, 0)),
            pl.BlockSpec((R, 1), lambda i: (i, 0)),
            pl.BlockSpec((d_in, d_hid), lambda i: (0, 0)),
            pl.BlockSpec((1, d_hid), lambda i: (0, 0)),
            pl.BlockSpec((d_hid, DP), lambda i: (0, 0)),
        ],
        out_specs=pl.BlockSpec((R, DP), lambda i: (i, 0)),
        out_shape=jax.ShapeDtypeStruct((NP, DP), jnp.float32),
    )(parts1, xp, dinv, W1, b1.reshape(1, d_hid), W2p)

    parts2 = _scatter128(zp, src_p, dst_p)

    out_pad = pl.pallas_call(
        _tc3_body,
        grid=(NP // R,),
        in_specs=[
            pl.BlockSpec((NC, R, DP), lambda i: (0, i, 0)),
            pl.BlockSpec((R, DP), lambda i: (i, 0)),
            pl.BlockSpec((R, 1), lambda i: (i, 0)),
            pl.BlockSpec((1, d_out), lambda i: (0, 0)),
        ],
        out_specs=pl.BlockSpec((R, d_out), lambda i: (i, 0)),
        out_shape=jax.ShapeDtypeStruct((NP, d_out), jnp.float32),
    )(parts2, zp, dinv, b2.reshape(1, d_out))

    return out_pad[:N0]


# speedup vs baseline: 4.4382x; 1.4643x over previous
import functools

import jax
import jax.numpy as jnp
from jax import lax
from jax.experimental import pallas as pl
from jax.experimental.pallas import tpu as pltpu
from jax.experimental.pallas import tpu_sc as plsc

N0 = 10000
NP = 10240
E0 = 320000
B = 128
NW = 32
NS = 16
NC = 2
G = 80
EP = NW * G * B
RPT = NP // NS
NSL = 4

_mesh = functools.partial(
    plsc.VectorSubcoreMesh, core_axis_name="c", subcore_axis_name="s"
)


def _make_deg_kernel():
    @functools.partial(
        pl.kernel,
        mesh=_mesh(),
        out_type=jax.ShapeDtypeStruct((NC, NP), jnp.float32),
        scratch_types=[
            pltpu.VMEM((G, B), jnp.int32),
            pltpu.VMEM((128,), jnp.float32),
            pltpu.VMEM((B,), jnp.float32),
            pltpu.VMEM_SHARED((NP,), jnp.float32),
        ],
    )
    def deg_kernel(dst_hbm, out_hbm, dst_v, zbuf, ones_v, acc):
        c = lax.axis_index("c")
        s = lax.axis_index("s")
        wid = s * NC + c
        pltpu.sync_copy(dst_hbm.at[wid], dst_v)
        for i in range(8):
            zbuf[pl.ds(i * 16, 16)] = jnp.zeros((16,), jnp.float32)
        for i in range(B // 16):
            ones_v[pl.ds(i * 16, 16)] = jnp.ones((16,), jnp.float32)
        base = s * RPT
        for off in range(0, RPT, 128):
            pltpu.sync_copy(zbuf, acc.at[pl.ds(base + off, 128)])
        plsc.subcore_barrier()

        def body(g, carry):
            pltpu.sync_copy(ones_v, acc.at[dst_v.at[g]], add=True)
            return carry

        lax.fori_loop(0, G, body, None)
        plsc.subcore_barrier()
        for off in range(0, RPT, 128):
            pltpu.sync_copy(
                acc.at[pl.ds(base + off, 128)], out_hbm.at[c, pl.ds(base + off, 128)]
            )

    return deg_kernel


def _make_scatter_kernel(D):
    @functools.partial(
        pl.kernel,
        mesh=_mesh(),
        out_type=jax.ShapeDtypeStruct((NC, NP, D), jnp.float32),
        scratch_types=[
            pltpu.VMEM((NSL, 2, B), jnp.int32),
            pltpu.VMEM((B, D), jnp.float32),
            pltpu.VMEM((B, D), jnp.float32),
            pltpu.VMEM_SHARED((NP, D), jnp.float32),
            pltpu.SemaphoreType.DMA,
            pltpu.SemaphoreType.DMA,
            pltpu.SemaphoreType.DMA,
            pltpu.SemaphoreType.DMA,
            pltpu.SemaphoreType.DMA,
            pltpu.SemaphoreType.DMA,
        ],
    )
    def scatter_kernel(
        table_hbm, sd_hbm, out_hbm,
        sdidx, rbuf0, rbuf1, acc,
        gsem0, gsem1, isem0, isem1, isem2, isem3,
    ):
        c = lax.axis_index("c")
        s = lax.axis_index("s")
        wid = s * NC + c
        rbufs = (rbuf0, rbuf1)
        gsems = (gsem0, gsem1)
        isems = (isem0, isem1, isem2, isem3)

        def zrow(r, carry):
            for i in range(D // 16):
                rbuf0[r, pl.ds(i * 16, 16)] = jnp.zeros((16,), jnp.float32)
            return carry

        lax.fori_loop(0, B, zrow, None)
        base = s * RPT
        for off in range(0, RPT, B):
            pltpu.sync_copy(rbuf0, acc.at[pl.ds(base + off, B)])
        plsc.subcore_barrier()

        def prefetch_idx(b2, g):
            pltpu.async_copy(sd_hbm.at[wid, g], sdidx.at[b2], isems[b2])

        def wait_idx(b2, g):
            pltpu.make_async_copy(sd_hbm.at[wid, g], sdidx.at[b2], isems[b2]).wait()

        for b2 in range(NSL):
            prefetch_idx(b2, b2)
        wait_idx(0, 0)
        pltpu.async_copy(table_hbm.at[sdidx.at[0, 0]], rbuf0, gsem0)

        def body(i, carry):
            for b in range(NSL):
                g = i * NSL + b
                nb = (b + 1) % NSL
                wait_idx(nb, g + 1)
                pltpu.async_copy(
                    table_hbm.at[sdidx.at[nb, 0]], rbufs[(b + 1) % 2], gsems[(b + 1) % 2]
                )
                pltpu.make_async_copy(
                    table_hbm.at[sdidx.at[b, 0]], rbufs[b % 2], gsems[b % 2]
                ).wait()
                pltpu.sync_copy(rbufs[b % 2], acc.at[sdidx.at[b, 1]], add=True)
                prefetch_idx(b, g + NSL)
            return carry

        lax.fori_loop(0, G // NSL, body, None)
        pltpu.make_async_copy(table_hbm.at[sdidx.at[0, 0]], rbuf0, gsem0).wait()
        for b2 in range(1, NSL):
            wait_idx(b2, G + b2)
        plsc.subcore_barrier()
        for off in range(0, RPT, B):
            pltpu.sync_copy(
                acc.at[pl.ds(base + off, B)],
                out_hbm.at[c, pl.ds(base + off, B)],
            )

    return scatter_kernel


_deg_kernel = _make_deg_kernel()
_scatter128 = _make_scatter_kernel(128)


def _tc1_body(degp_ref, x_ref, dinv_ref, xp_ref):
    deg = degp_ref[0] + degp_ref[1] + 1.0
    dinv = lax.rsqrt(deg)
    dinv_ref[...] = dinv
    xp_ref[...] = x_ref[...] * dinv


def _tc2_body(parts_ref, xp_ref, dinv_ref, w1_ref, b1_ref, w2_ref, zp_ref):
    dinv = dinv_ref[...]
    p = (parts_ref[0] + parts_ref[1] + xp_ref[...]) * dinv
    h = jnp.dot(p, w1_ref[...], preferred_element_type=jnp.float32) + b1_ref[...]
    h = jnp.maximum(h, 0.0)
    z = jnp.dot(h, w2_ref[...], preferred_element_type=jnp.float32)
    zp_ref[...] = z * dinv


def _tc3_body(parts_ref, zp_ref, dinv_ref, b2_ref, out_ref):
    d_out = out_ref.shape[1]
    out_ref[...] = (
        parts_ref[0][:, :d_out] + parts_ref[1][:, :d_out] + zp_ref[:, :d_out]
    ) * dinv_ref[...] + b2_ref[...]


def kernel(x, edge_index, W1, b1, W2, b2):
    d_in = x.shape[1]
    d_hid = W1.shape[1]
    d_out = W2.shape[1]

    pad = EP - E0
    pad_idx = N0 + (jnp.arange(pad, dtype=jnp.int32) % (NP - N0))
    src_p = jnp.concatenate([edge_index[0], pad_idx]).reshape(NW, G, B)
    dst_p = jnp.concatenate([edge_index[1], pad_idx]).reshape(NW, G, B)
    extra = N0 + (
        jnp.arange(NW * NSL * 2 * B, dtype=jnp.int32).reshape(NW, NSL, 2, B)
        % (NP - N0)
    )
    sd_p = jnp.concatenate([jnp.stack([src_p, dst_p], axis=2), extra], axis=1)
    x_pad = jnp.concatenate([x, jnp.zeros((NP - N0, d_in), x.dtype)])

    deg_parts = _deg_kernel(dst_p)

    dinv, xp = pl.pallas_call(
        _tc1_body,
        out_shape=(
            jax.ShapeDtypeStruct((NP, 1), jnp.float32),
            jax.ShapeDtypeStruct((NP, d_in), jnp.float32),
        ),
    )(deg_parts.reshape(NC, NP, 1), x_pad)

    parts1 = _scatter128(xp, sd_p)

    DP = 128
    W2p = jnp.concatenate([W2, jnp.zeros((d_hid, DP - d_out), W2.dtype)], axis=1)
    R = 1024
    zp = pl.pallas_call(
        _tc2_body,
        grid=(NP // R,),
        in_specs=[
            pl.BlockSpec((NC, R, d_in), lambda i: (0, i, 0)),
            pl.BlockSpec((R, d_in), lambda i: (i, 0)),
            pl.BlockSpec((R, 1), lambda i: (i, 0)),
            pl.BlockSpec((d_in, d_hid), lambda i: (0, 0)),
            pl.BlockSpec((1, d_hid), lambda i: (0, 0)),
            pl.BlockSpec((d_hid, DP), lambda i: (0, 0)),
        ],
        out_specs=pl.BlockSpec((R, DP), lambda i: (i, 0)),
        out_shape=jax.ShapeDtypeStruct((NP, DP), jnp.float32),
    )(parts1, xp, dinv, W1, b1.reshape(1, d_hid), W2p)

    parts2 = _scatter128(zp, sd_p)

    out_pad = pl.pallas_call(
        _tc3_body,
        grid=(NP // R,),
        in_specs=[
            pl.BlockSpec((NC, R, DP), lambda i: (0, i, 0)),
            pl.BlockSpec((R, DP), lambda i: (i, 0)),
            pl.BlockSpec((R, 1), lambda i: (i, 0)),
            pl.BlockSpec((1, d_out), lambda i: (0, 0)),
        ],
        out_specs=pl.BlockSpec((R, d_out), lambda i: (i, 0)),
        out_shape=jax.ShapeDtypeStruct((NP, d_out), jnp.float32),
    )(parts2, zp, dinv, b2.reshape(1, d_out))

    return out_pad[:N0]


# speedup vs baseline: 4.6990x; 1.0588x over previous
import functools

import jax
import jax.numpy as jnp
from jax import lax
from jax.experimental import pallas as pl
from jax.experimental.pallas import tpu as pltpu
from jax.experimental.pallas import tpu_sc as plsc

N0 = 10000
NP = 10240
E0 = 320000
B = 128
NW = 32
NS = 16
NC = 2
G = 80
EP = NW * G * B
RPT = NP // NS
NSL = 4

_mesh = functools.partial(
    plsc.VectorSubcoreMesh, core_axis_name="c", subcore_axis_name="s"
)


def _make_deg_kernel():
    @functools.partial(
        pl.kernel,
        mesh=_mesh(),
        out_type=jax.ShapeDtypeStruct((NC, NP), jnp.float32),
        scratch_types=[
            pltpu.VMEM((G, B), jnp.int32),
            pltpu.VMEM((128,), jnp.float32),
            pltpu.VMEM((B,), jnp.float32),
            pltpu.VMEM_SHARED((NP,), jnp.float32),
        ],
    )
    def deg_kernel(dst_hbm, out_hbm, dst_v, zbuf, ones_v, acc):
        c = lax.axis_index("c")
        s = lax.axis_index("s")
        wid = s * NC + c
        pltpu.sync_copy(dst_hbm.at[wid], dst_v)
        for i in range(8):
            zbuf[pl.ds(i * 16, 16)] = jnp.zeros((16,), jnp.float32)
        for i in range(B // 16):
            ones_v[pl.ds(i * 16, 16)] = jnp.ones((16,), jnp.float32)
        base = s * RPT
        for off in range(0, RPT, 128):
            pltpu.sync_copy(zbuf, acc.at[pl.ds(base + off, 128)])
        plsc.subcore_barrier()

        def body(g, carry):
            pltpu.sync_copy(ones_v, acc.at[dst_v.at[g]], add=True)
            return carry

        lax.fori_loop(0, G, body, None)
        plsc.subcore_barrier()
        for off in range(0, RPT, 128):
            pltpu.sync_copy(
                acc.at[pl.ds(base + off, 128)], out_hbm.at[c, pl.ds(base + off, 128)]
            )

    return deg_kernel


def _make_scatter_kernel(D, tc_tiling=True):
    @functools.partial(
        pl.kernel,
        mesh=_mesh(),
        out_type=jax.ShapeDtypeStruct((NC, NP, D), jnp.float32),
        compiler_params=pltpu.CompilerParams(use_tc_tiling_on_sc=tc_tiling),
        scratch_types=[
            pltpu.VMEM((NSL, 2, B), jnp.int32),
            pltpu.VMEM((B, D), jnp.float32),
            pltpu.VMEM((B, D), jnp.float32),
            pltpu.VMEM_SHARED((NP, D), jnp.float32),
            pltpu.SemaphoreType.DMA,
            pltpu.SemaphoreType.DMA,
            pltpu.SemaphoreType.DMA,
            pltpu.SemaphoreType.DMA,
            pltpu.SemaphoreType.DMA,
            pltpu.SemaphoreType.DMA,
        ],
    )
    def scatter_kernel(
        table_hbm, sd_hbm, out_hbm,
        sdidx, rbuf0, rbuf1, acc,
        gsem0, gsem1, isem0, isem1, isem2, isem3,
    ):
        c = lax.axis_index("c")
        s = lax.axis_index("s")
        wid = s * NC + c
        rbufs = (rbuf0, rbuf1)
        gsems = (gsem0, gsem1)
        isems = (isem0, isem1, isem2, isem3)

        def zrow(r, carry):
            for i in range(D // 16):
                rbuf0[r, pl.ds(i * 16, 16)] = jnp.zeros((16,), jnp.float32)
            return carry

        lax.fori_loop(0, B, zrow, None)
        base = s * RPT
        for off in range(0, RPT, B):
            pltpu.sync_copy(rbuf0, acc.at[pl.ds(base + off, B)])
        plsc.subcore_barrier()

        def prefetch_idx(b2, g):
            pltpu.async_copy(sd_hbm.at[wid, g], sdidx.at[b2], isems[b2])

        def wait_idx(b2, g):
            pltpu.make_async_copy(sd_hbm.at[wid, g], sdidx.at[b2], isems[b2]).wait()

        for b2 in range(NSL):
            prefetch_idx(b2, b2)
        wait_idx(0, 0)
        pltpu.async_copy(table_hbm.at[sdidx.at[0, 0]], rbuf0, gsem0)

        def body(i, carry):
            for b in range(NSL):
                g = i * NSL + b
                nb = (b + 1) % NSL
                wait_idx(nb, g + 1)
                pltpu.async_copy(
                    table_hbm.at[sdidx.at[nb, 0]], rbufs[(b + 1) % 2], gsems[(b + 1) % 2]
                )
                pltpu.make_async_copy(
                    table_hbm.at[sdidx.at[b, 0]], rbufs[b % 2], gsems[b % 2]
                ).wait()
                pltpu.sync_copy(rbufs[b % 2], acc.at[sdidx.at[b, 1]], add=True)
                prefetch_idx(b, g + NSL)
            return carry

        lax.fori_loop(0, G // NSL, body, None)
        pltpu.make_async_copy(table_hbm.at[sdidx.at[0, 0]], rbuf0, gsem0).wait()
        for b2 in range(1, NSL):
            wait_idx(b2, G + b2)
        plsc.subcore_barrier()
        for off in range(0, RPT, B):
            pltpu.sync_copy(
                acc.at[pl.ds(base + off, B)],
                out_hbm.at[c, pl.ds(base + off, B)],
            )

    return scatter_kernel


_deg_kernel = _make_deg_kernel()
_scatter128 = _make_scatter_kernel(128)
_scatter64 = _make_scatter_kernel(64, tc_tiling=False)


def _tc1_body(degp_ref, x_ref, dinv_ref, xp_ref):
    deg = degp_ref[0] + degp_ref[1] + 1.0
    dinv = lax.rsqrt(deg)
    dinv_ref[...] = dinv
    xp_ref[...] = x_ref[...] * dinv


def _tc2_body(parts_ref, xp_ref, dinv_ref, w1_ref, b1_ref, w2_ref, zp_ref):
    dinv = dinv_ref[...]
    p = (parts_ref[0] + parts_ref[1] + xp_ref[...]) * dinv
    h = jnp.dot(p, w1_ref[...], preferred_element_type=jnp.float32) + b1_ref[...]
    h = jnp.maximum(h, 0.0)
    z = jnp.dot(h, w2_ref[...], preferred_element_type=jnp.float32)
    zp_ref[...] = z * dinv


def _tc3_body(parts_ref, zp_ref, dinv_ref, b2_ref, out_ref):
    d_out = out_ref.shape[1]
    out_ref[...] = (
        parts_ref[0][:, :d_out] + parts_ref[1][:, :d_out] + zp_ref[:, :d_out]
    ) * dinv_ref[...] + b2_ref[...]


def kernel(x, edge_index, W1, b1, W2, b2):
    d_in = x.shape[1]
    d_hid = W1.shape[1]
    d_out = W2.shape[1]

    pad = EP - E0
    pad_idx = N0 + (jnp.arange(pad, dtype=jnp.int32) % (NP - N0))
    src_p = jnp.concatenate([edge_index[0], pad_idx]).reshape(NW, G, B)
    dst_p = jnp.concatenate([edge_index[1], pad_idx]).reshape(NW, G, B)
    extra = N0 + (
        jnp.arange(NW * NSL * 2 * B, dtype=jnp.int32).reshape(NW, NSL, 2, B)
        % (NP - N0)
    )
    sd_p = jnp.concatenate([jnp.stack([src_p, dst_p], axis=2), extra], axis=1)
    x_pad = jnp.concatenate([x, jnp.zeros((NP - N0, d_in), x.dtype)])

    deg_parts = _deg_kernel(dst_p)

    dinv, xp = pl.pallas_call(
        _tc1_body,
        out_shape=(
            jax.ShapeDtypeStruct((NP, 1), jnp.float32),
            jax.ShapeDtypeStruct((NP, d_in), jnp.float32),
        ),
    )(deg_parts.reshape(NC, NP, 1), x_pad)

    parts1 = _scatter128(xp, sd_p)

    R = 1024
    zp = pl.pallas_call(
        _tc2_body,
        grid=(NP // R,),
        in_specs=[
            pl.BlockSpec((NC, R, d_in), lambda i: (0, i, 0)),
            pl.BlockSpec((R, d_in), lambda i: (i, 0)),
            pl.BlockSpec((R, 1), lambda i: (i, 0)),
            pl.BlockSpec((d_in, d_hid), lambda i: (0, 0)),
            pl.BlockSpec((1, d_hid), lambda i: (0, 0)),
            pl.BlockSpec((d_hid, d_out), lambda i: (0, 0)),
        ],
        out_specs=pl.BlockSpec((R, d_out), lambda i: (i, 0)),
        out_shape=jax.ShapeDtypeStruct((NP, d_out), jnp.float32),
    )(parts1, xp, dinv, W1, b1.reshape(1, d_hid), W2)

    parts2 = _scatter64(zp, sd_p)

    R3 = 400
    out = pl.pallas_call(
        _tc3_body,
        grid=(N0 // R3,),
        in_specs=[
            pl.BlockSpec((NC, R3, d_out), lambda i: (0, i, 0)),
            pl.BlockSpec((R3, d_out), lambda i: (i, 0)),
            pl.BlockSpec((R3, 1), lambda i: (i, 0)),
            pl.BlockSpec((1, d_out), lambda i: (0, 0)),
        ],
        out_specs=pl.BlockSpec((R3, d_out), lambda i: (i, 0)),
        out_shape=jax.ShapeDtypeStruct((N0, d_out), jnp.float32),
    )(parts2, zp, dinv, b2.reshape(1, d_out))

    return out
